# Initial kernel scaffold; baseline (speedup 1.0000x reference)
import jax
import jax.numpy as jnp
from jax.experimental import pallas as pl


def kernel(nfeats, efeats, edge_index, W_msg, b_msg, W_apply, b_apply):
    raise NotImplementedError("write your pallas kernel here")


# speedup vs baseline: 2.2016x; 2.2016x over previous
import functools

import jax
import jax.numpy as jnp
from jax import lax
from jax.experimental import pallas as pl
from jax.experimental.pallas import tpu as pltpu
from jax.experimental.pallas import tpu_sc as plsc

N = 10000
E = 320000
D = 128
DE = 16

NC = 2
NS = 16
NW = NC * NS
ECHUNK = 128
CPW = 79
EPW = CPW * ECHUNK
EPAD = NW * EPW
RPT = 640
NSH = NS * RPT


def _z_body(nf_ref, w_ref, o_ref):
    o_ref[...] = jnp.dot(nf_ref[...], w_ref[...],
                         preferred_element_type=jnp.float32)


def _compute_z(nfeats, W1):
    blk = 400
    return pl.pallas_call(
        _z_body,
        grid=(N // blk,),
        in_specs=[
            pl.BlockSpec((blk, D), lambda i: (i, 0)),
            pl.BlockSpec((D, D), lambda i: (0, 0)),
        ],
        out_specs=pl.BlockSpec((blk, D), lambda i: (i, 0)),
        out_shape=jax.ShapeDtypeStruct((N, D), jnp.float32),
    )(nfeats, W1)


def _w_body(ef_ref, w_ref, b_ref, o_ref):
    o_ref[...] = jnp.dot(ef_ref[...], w_ref[...],
                         preferred_element_type=jnp.float32) + b_ref[...]


def _compute_w(efeats_pad, W2, b_msg):
    blk = 2048
    return pl.pallas_call(
        _w_body,
        grid=(EPAD // blk,),
        in_specs=[
            pl.BlockSpec((blk, DE), lambda i: (i, 0)),
            pl.BlockSpec((DE, D), lambda i: (0, 0)),
            pl.BlockSpec((1, D), lambda i: (0, 0)),
        ],
        out_specs=pl.BlockSpec((blk, D), lambda i: (i, 0)),
        out_shape=jax.ShapeDtypeStruct((EPAD, D), jnp.float32),
    )(efeats_pad, W2, b_msg.reshape(1, D))


def _sc_body(src_hbm, dst_hbm, z_hbm, w_hbm, zeros_hbm, out_hbm,
             src_v, dst_v, zrow_v, w_v, acc_sh, sem):
    c = lax.axis_index("c")
    s = lax.axis_index("s")
    wid = s * NC + c
    base0 = wid * EPW
    tile_rows = s * RPT

    pltpu.sync_copy(zeros_hbm, acc_sh.at[pl.ds(tile_rows, RPT)])
    plsc.subcore_barrier()

    def chunk(k, carry):
        base = base0 + k * ECHUNK
        pltpu.sync_copy(src_hbm.at[pl.ds(base, ECHUNK)], src_v)
        pltpu.sync_copy(dst_hbm.at[pl.ds(base, ECHUNK)], dst_v)
        pltpu.async_copy(z_hbm.at[src_v], zrow_v, sem).wait()
        pltpu.sync_copy(w_hbm.at[pl.ds(base, ECHUNK)], w_v)

        def row(i, carry2):
            for j in range(D // 16):
                sl = pl.ds(j * 16, 16)
                w_v[i, sl] = jnp.maximum(w_v[i, sl] + zrow_v[i, sl], 0.0)
            return carry2

        lax.fori_loop(0, ECHUNK, row, 0)
        pltpu.sync_copy(w_v, acc_sh.at[dst_v], add=True)
        return carry

    lax.fori_loop(0, CPW, chunk, 0)
    plsc.subcore_barrier()
    pltpu.sync_copy(acc_sh.at[pl.ds(tile_rows, RPT)],
                    out_hbm.at[c, pl.ds(tile_rows, RPT)])


_sc_aggregate = functools.partial(
    pl.kernel,
    out_type=jax.ShapeDtypeStruct((NC, NSH, D), jnp.float32),
    mesh=plsc.VectorSubcoreMesh(core_axis_name="c", subcore_axis_name="s"),
    scratch_types=[
        pltpu.VMEM((ECHUNK,), jnp.int32),
        pltpu.VMEM((ECHUNK,), jnp.int32),
        pltpu.VMEM((ECHUNK, D), jnp.float32),
        pltpu.VMEM((ECHUNK, D), jnp.float32),
        pltpu.VMEM_SHARED((NSH, D), jnp.float32),
        pltpu.SemaphoreType.DMA,
    ],
)(_sc_body)


def _apply_body(nf_ref, p0_ref, p1_ref, wa1_ref, wa2_ref, b_ref, o_ref):
    hn = p0_ref[...] + p1_ref[...]
    acc = jnp.dot(nf_ref[...], wa1_ref[...], preferred_element_type=jnp.float32)
    acc += jnp.dot(hn, wa2_ref[...], preferred_element_type=jnp.float32)
    o_ref[...] = jnp.maximum(acc + b_ref[...], 0.0)


def _apply(nfeats, p0, p1, Wa1, Wa2, b_apply):
    blk = 400
    return pl.pallas_call(
        _apply_body,
        grid=(N // blk,),
        in_specs=[
            pl.BlockSpec((blk, D), lambda i: (i, 0)),
            pl.BlockSpec((blk, D), lambda i: (i, 0)),
            pl.BlockSpec((blk, D), lambda i: (i, 0)),
            pl.BlockSpec((D, D), lambda i: (0, 0)),
            pl.BlockSpec((D, D), lambda i: (0, 0)),
            pl.BlockSpec((1, D), lambda i: (0, 0)),
        ],
        out_specs=pl.BlockSpec((blk, D), lambda i: (i, 0)),
        out_shape=jax.ShapeDtypeStruct((N, D), jnp.float32),
    )(nfeats, p0, p1, Wa1, Wa2, b_apply.reshape(1, D))


def kernel(nfeats, efeats, edge_index, W_msg, b_msg, W_apply, b_apply):
    src = edge_index[0].astype(jnp.int32)
    dst = edge_index[1].astype(jnp.int32)
    npad = EPAD - E
    src_pad = jnp.concatenate([src, jnp.zeros((npad,), jnp.int32)])
    dst_pad = jnp.concatenate([dst, jnp.full((npad,), N, jnp.int32)])
    efeats_pad = jnp.concatenate(
        [efeats, jnp.zeros((npad, DE), jnp.float32)], axis=0)

    z = _compute_z(nfeats, W_msg[:D])
    w = _compute_w(efeats_pad, W_msg[D:], b_msg)
    zeros = jnp.zeros((RPT, D), jnp.float32)
    partials = _sc_aggregate(src_pad, dst_pad, z, w, zeros)
    return _apply(nfeats, partials[0, :N], partials[1, :N],
                  W_apply[:D], W_apply[D:], b_apply)


# speedup vs baseline: 2.3653x; 1.0744x over previous
import functools

import jax
import jax.numpy as jnp
from jax import lax
from jax.experimental import pallas as pl
from jax.experimental.pallas import tpu as pltpu
from jax.experimental.pallas import tpu_sc as plsc

N = 10000
E = 320000
D = 128
DE = 16

NC = 2
NS = 16
NW = NC * NS
ECHUNK = 64
CPW = 160
EPW = CPW * ECHUNK
EPAD = NW * EPW
RPT = 640
NSH = NS * RPT


def _z_body(nf_ref, w_ref, o_ref):
    o_ref[...] = jnp.dot(nf_ref[...], w_ref[...],
                         preferred_element_type=jnp.float32)


def _compute_z(nfeats, W1):
    blk = 400
    return pl.pallas_call(
        _z_body,
        grid=(N // blk,),
        in_specs=[
            pl.BlockSpec((blk, D), lambda i: (i, 0)),
            pl.BlockSpec((D, D), lambda i: (0, 0)),
        ],
        out_specs=pl.BlockSpec((blk, D), lambda i: (i, 0)),
        out_shape=jax.ShapeDtypeStruct((N, D), jnp.float32),
    )(nfeats, W1)


def _w_body(ef_ref, w_ref, b_ref, o_ref):
    o_ref[...] = jnp.dot(ef_ref[...], w_ref[...],
                         preferred_element_type=jnp.float32) + b_ref[...]


def _compute_w(efeats, W2, b_msg):
    blk = 2048
    last = (E - 1) // blk
    return pl.pallas_call(
        _w_body,
        grid=(EPAD // blk,),
        in_specs=[
            pl.BlockSpec((blk, DE), lambda i: (jnp.minimum(i, last), 0)),
            pl.BlockSpec((DE, D), lambda i: (0, 0)),
            pl.BlockSpec((1, D), lambda i: (0, 0)),
        ],
        out_specs=pl.BlockSpec((blk, D), lambda i: (i, 0)),
        out_shape=jax.ShapeDtypeStruct((EPAD, D), jnp.float32),
    )(efeats, W2, b_msg.reshape(1, D))


def _sc_body(idx_hbm, z_hbm, w_hbm, zeros_hbm, out_hbm,
             idxq_v, z0, z1, w0, w1, acc_sh,
             gsem0, gsem1, wsem0, wsem1, ssem0, ssem1,
             isem0, isem1, isem2, isem3):
    c = lax.axis_index("c")
    s = lax.axis_index("s")
    wid = s * NC + c
    wbase = wid * EPW
    tile_rows = s * RPT

    zbuf = (z0, z1)
    wbuf = (w0, w1)
    gsem = (gsem0, gsem1)
    wsem = (wsem0, wsem1)
    ssem = (ssem0, ssem1)
    isem = (isem0, isem1, isem2, isem3)

    pltpu.sync_copy(zeros_hbm, acc_sh.at[pl.ds(tile_rows, RPT)])
    plsc.subcore_barrier()

    def issue_src(k, q):
        pltpu.async_copy(idx_hbm.at[wid, k], idxq_v.at[q], isem[q])

    def wait_src(k, q):
        pltpu.make_async_copy(idx_hbm.at[wid, k], idxq_v.at[q],
                              isem[q]).wait()

    def issue_gather(k, q, b):
        pltpu.async_copy(z_hbm.at[idxq_v.at[q, 0]], zbuf[b], gsem[b])

    def wait_gather(k, q, b):
        pltpu.make_async_copy(z_hbm.at[idxq_v.at[q, 0]], zbuf[b],
                              gsem[b]).wait()

    def issue_w(k, b):
        pltpu.async_copy(w_hbm.at[pl.ds(wbase + k * ECHUNK, ECHUNK)],
                         wbuf[b], wsem[b])

    def wait_w(k, b):
        pltpu.make_async_copy(
            w_hbm.at[pl.ds(wbase + k * ECHUNK, ECHUNK)],
            wbuf[b], wsem[b]).wait()

    def compute(b):
        zb, wb = zbuf[b], wbuf[b]

        @plsc.parallel_loop(0, ECHUNK, unroll=4)
        def _row(r):
            for j in range(D // 16):
                sl = pl.ds(j * 16, 16)
                wb[r, sl] = jnp.maximum(wb[r, sl] + zb[r, sl], 0.0)

    def issue_scatter(q, b):
        pltpu.async_copy(wbuf[b], acc_sh.at[idxq_v.at[q, 1]], ssem[b],
                         add=True)

    def wait_scatter(q, b):
        pltpu.make_async_copy(wbuf[b], acc_sh.at[idxq_v.at[q, 1]],
                              ssem[b]).wait()

    def step(k, j, first, last):
        b, bn = j % 2, (j + 1) % 2
        wait_gather(k, j, b)
        wait_w(k, b)
        if not last:
            wait_src(k + 1, (j + 1) % 4)
            issue_gather(k + 1, (j + 1) % 4, bn)
            issue_src(k + 2, (j + 2) % 4)
        compute(b)
        if not first:
            wait_scatter((j + 3) % 4, bn)
        if not last:
            issue_w(k + 1, bn)
        issue_scatter(j, b)

    issue_src(0, 0)
    issue_src(1, 1)
    wait_src(0, 0)
    issue_gather(0, 0, 0)
    issue_w(0, 0)

    for j in range(4):
        step(j, j, first=(j == 0), last=False)

    def body(g, carry):
        for j in range(4):
            step(4 * g + j, j, first=False, last=False)
        return carry

    lax.fori_loop(1, CPW // 4 - 1, body, 0)

    base = CPW - 4
    for j in range(4):
        k = base + j
        b, bn = j % 2, (j + 1) % 2
        wait_gather(k, j, b)
        wait_w(k, b)
        if j < 3:
            wait_src(k + 1, (j + 1) % 4)
            issue_gather(k + 1, (j + 1) % 4, bn)
        if j < 2:
            issue_src(k + 2, (j + 2) % 4)
        compute(b)
        wait_scatter((j + 3) % 4, bn)
        if j < 3:
            issue_w(k + 1, bn)
        issue_scatter(j, b)
    wait_scatter(3, (CPW - 1) % 2)

    plsc.subcore_barrier()
    pltpu.sync_copy(acc_sh.at[pl.ds(tile_rows, RPT)],
                    out_hbm.at[c, pl.ds(tile_rows, RPT)])


_sc_aggregate = functools.partial(
    pl.kernel,
    out_type=jax.ShapeDtypeStruct((NC, NSH, D), jnp.float32),
    mesh=plsc.VectorSubcoreMesh(core_axis_name="c", subcore_axis_name="s"),
    scratch_types=(
        [pltpu.VMEM((4, 2, ECHUNK), jnp.int32)]
        + [pltpu.VMEM((ECHUNK, D), jnp.float32) for _ in range(4)]
        + [pltpu.VMEM_SHARED((NSH, D), jnp.float32)]
        + [pltpu.SemaphoreType.DMA for _ in range(10)]
    ),
)(_sc_body)


def _apply_body(nf_ref, p0_ref, p1_ref, wa1_ref, wa2_ref, b_ref, o_ref):
    hn = p0_ref[...] + p1_ref[...]
    acc = jnp.dot(nf_ref[...], wa1_ref[...], preferred_element_type=jnp.float32)
    acc += jnp.dot(hn, wa2_ref[...], preferred_element_type=jnp.float32)
    o_ref[...] = jnp.maximum(acc + b_ref[...], 0.0)


def _apply(nfeats, p0, p1, Wa1, Wa2, b_apply):
    blk = 400
    return pl.pallas_call(
        _apply_body,
        grid=(N // blk,),
        in_specs=[
            pl.BlockSpec((blk, D), lambda i: (i, 0)),
            pl.BlockSpec((blk, D), lambda i: (i, 0)),
            pl.BlockSpec((blk, D), lambda i: (i, 0)),
            pl.BlockSpec((D, D), lambda i: (0, 0)),
            pl.BlockSpec((D, D), lambda i: (0, 0)),
            pl.BlockSpec((1, D), lambda i: (0, 0)),
        ],
        out_specs=pl.BlockSpec((blk, D), lambda i: (i, 0)),
        out_shape=jax.ShapeDtypeStruct((N, D), jnp.float32),
    )(nfeats, p0, p1, Wa1, Wa2, b_apply.reshape(1, D))


def kernel(nfeats, efeats, edge_index, W_msg, b_msg, W_apply, b_apply):
    src = edge_index[0].astype(jnp.int32)
    dst = edge_index[1].astype(jnp.int32)
    npad = EPAD - E
    src_pad = jnp.concatenate([src, jnp.zeros((npad,), jnp.int32)])
    dump = N + (jnp.arange(npad, dtype=jnp.int32) % (NSH - N))
    dst_pad = jnp.concatenate([dst, dump])
    idx_pad = jnp.stack([src_pad.reshape(NW, CPW, ECHUNK),
                         dst_pad.reshape(NW, CPW, ECHUNK)], axis=2)

    z = _compute_z(nfeats, W_msg[:D])
    w = _compute_w(efeats, W_msg[D:], b_msg)
    zeros = jnp.zeros((RPT, D), jnp.float32)
    partials = _sc_aggregate(idx_pad, z, w, zeros)
    return _apply(nfeats, partials[0, :N], partials[1, :N],
                  W_apply[:D], W_apply[D:], b_apply)
